# baseline (speedup 1.0000x reference)
import functools

import jax
import jax.numpy as jnp
from jax import lax
from jax.experimental import pallas as pl
from jax.experimental.pallas import tpu as pltpu
from jax.experimental.pallas import tpu_sc as plsc

BATCH = 16384
DIM = 64
LANES = 16
GRPS = DIM // LANES

_info = plsc.get_sparse_core_info()
NC, NS = _info.num_cores, _info.num_subcores
NW = NC * NS
PER_W = BATCH // NW
CHUNK = 128
NCHUNK = PER_W // CHUNK
KIDX = 128
NK = CHUNK // KIDX


def _tec_body(head_hbm, rel_hbm, tail_hbm, ent2_hbm, rel_cat_hbm, out_hbm,
              hidx, tidx, ridx, hbase, tbase, h_v, t_v, rel_v, out_v, sem):
    wid = lax.axis_index("s") * NC + lax.axis_index("c")
    wbase = wid * PER_W
    lane = lax.iota(jnp.int32, LANES)
    perms = [jnp.bitwise_xor(lane, s) for s in (1, 2, 4, 8)]

    for c in range(NCHUNK):
        base = wbase + c * CHUNK
        for k in range(NK):
            off = base + k * KIDX
            pltpu.sync_copy(head_hbm.at[pl.ds(off, KIDX)], hidx.at[k])
            pltpu.sync_copy(tail_hbm.at[pl.ds(off, KIDX)], tidx.at[k])
            pltpu.sync_copy(rel_hbm.at[pl.ds(off, KIDX)], ridx.at[k])
        for k in range(NK):
            for v in range(KIDX // LANES):
                sl = pl.ds(v * LANES, LANES)
                dst = pl.ds(k * KIDX + v * LANES, LANES)
                hk = hidx[k, sl]
                tk = tidx[k, sl]
                hbase[dst] = (hk & 1) * DIM
                tbase[dst] = (tk & 1) * DIM
                hidx[k, sl] = jnp.right_shift(hk, 1)
                tidx[k, sl] = jnp.right_shift(tk, 1)
        cps = []
        for k in range(NK):
            dst = pl.ds(k * KIDX, KIDX)
            cps.append(pltpu.async_copy(ent2_hbm.at[hidx.at[k]],
                                        h_v.at[dst], sem))
            cps.append(pltpu.async_copy(ent2_hbm.at[tidx.at[k]],
                                        t_v.at[dst], sem))
            cps.append(pltpu.async_copy(rel_cat_hbm.at[ridx.at[k]],
                                        rel_v.at[dst], sem))
        for cp in cps:
            cp.wait()

        def row(i, _):
            bcast = jnp.full((LANES,), i, jnp.int32)
            hb = plsc.load_gather(hbase, [bcast])
            tb = plsc.load_gather(tbase, [bcast])
            u = []
            rh = []
            for j in range(GRPS):
                col = lane + j * LANES
                h_j = plsc.load_gather(h_v, [bcast, hb + col])
                t_j = plsc.load_gather(t_v, [bcast, tb + col])
                u.append(h_j - t_j)
                rh.append(rel_v[i, pl.ds(DIM + j * LANES, LANES)])
            acc = u[0] * rh[0]
            for j in range(1, GRPS):
                acc = acc + u[j] * rh[j]
            for p in perms:
                acc = acc + acc.at[p].get(mode="promise_in_bounds")
            for j in range(GRPS):
                sl = pl.ds(j * LANES, LANES)
                out_v[i, sl] = u[j] + rel_v[i, sl] - rh[j] * acc
            return 0

        lax.fori_loop(0, CHUNK, row, 0)
        pltpu.sync_copy(out_v, out_hbm.at[pl.ds(base, CHUNK)])


@jax.jit
def kernel(head, relation, tail, ent_emb, rel_emb, rel_hyper):
    ent2 = ent_emb.reshape(500000, 2 * DIM)
    rel_cat = jnp.concatenate([rel_emb, rel_hyper], axis=1)
    mesh = plsc.VectorSubcoreMesh(core_axis_name="c", subcore_axis_name="s")
    run = functools.partial(
        pl.kernel,
        mesh=mesh,
        out_type=jax.ShapeDtypeStruct((BATCH, DIM), jnp.float32),
        scratch_types=[
            pltpu.VMEM((NK, KIDX), jnp.int32),
            pltpu.VMEM((NK, KIDX), jnp.int32),
            pltpu.VMEM((NK, KIDX), jnp.int32),
            pltpu.VMEM((CHUNK,), jnp.int32),
            pltpu.VMEM((CHUNK,), jnp.int32),
            pltpu.VMEM((CHUNK, 2 * DIM), jnp.float32),
            pltpu.VMEM((CHUNK, 2 * DIM), jnp.float32),
            pltpu.VMEM((CHUNK, 2 * DIM), jnp.float32),
            pltpu.VMEM((CHUNK, DIM), jnp.float32),
            pltpu.SemaphoreType.DMA,
        ],
        compiler_params=pltpu.CompilerParams(use_tc_tiling_on_sc=True,
                                             needs_layout_passes=False),
    )(_tec_body)
    return run(head.astype(jnp.int32), relation.astype(jnp.int32),
               tail.astype(jnp.int32), ent2, rel_cat)


# speedup vs baseline: 1.0141x; 1.0141x over previous
import functools

import jax
import jax.numpy as jnp
from jax import lax
from jax.experimental import pallas as pl
from jax.experimental.pallas import tpu as pltpu
from jax.experimental.pallas import tpu_sc as plsc

BATCH = 16384
DIM = 64
LANES = 16
GRPS = DIM // LANES

_info = plsc.get_sparse_core_info()
NC, NS = _info.num_cores, _info.num_subcores
NW = NC * NS
PER_W = BATCH // NW
CHUNK = 128
NCHUNK = PER_W // CHUNK
KIDX = 128
NK = CHUNK // KIDX


def _tec_body(head_hbm, rel_hbm, tail_hbm, ent2_hbm, rel_cat_hbm, out_hbm,
              hidx, tidx, ridx, hbase, tbase, h_v, t_v, rel_v, out_v, sem):
    wid = lax.axis_index("s") * NC + lax.axis_index("c")
    wbase = wid * PER_W
    lane = lax.iota(jnp.int32, LANES)
    perms = [jnp.bitwise_xor(lane, s) for s in (1, 2, 4, 8)]
    ent_f = ent2_hbm.bitcast(jnp.float32)

    for c in range(NCHUNK):
        base = wbase + c * CHUNK
        for k in range(NK):
            off = base + k * KIDX
            pltpu.sync_copy(head_hbm.at[pl.ds(off, KIDX)], hidx.at[k])
            pltpu.sync_copy(tail_hbm.at[pl.ds(off, KIDX)], tidx.at[k])
            pltpu.sync_copy(rel_hbm.at[pl.ds(off, KIDX)], ridx.at[k])
        for k in range(NK):
            for v in range(KIDX // LANES):
                sl = pl.ds(v * LANES, LANES)
                dst = pl.ds(k * KIDX + v * LANES, LANES)
                hk = hidx[k, sl]
                tk = tidx[k, sl]
                hbase[dst] = (hk & 1) * DIM + jnp.right_shift(hk & 2, 1)
                tbase[dst] = (tk & 1) * DIM + jnp.right_shift(tk & 2, 1)
                hidx[k, sl] = jnp.right_shift(hk, 2)
                tidx[k, sl] = jnp.right_shift(tk, 2)
        cps = []
        for k in range(NK):
            dst = pl.ds(k * KIDX, KIDX)
            cps.append(pltpu.async_copy(ent_f.at[hidx.at[k]],
                                        h_v.at[dst], sem))
            cps.append(pltpu.async_copy(ent_f.at[tidx.at[k]],
                                        t_v.at[dst], sem))
            cps.append(pltpu.async_copy(rel_cat_hbm.at[ridx.at[k]],
                                        rel_v.at[dst], sem))
        for cp in cps:
            cp.wait()

        def row(i, _):
            bcast = jnp.full((LANES,), i, jnp.int32)
            hbt = plsc.load_gather(hbase, [bcast])
            tbt = plsc.load_gather(tbase, [bcast])
            hb = hbt & ~1
            tb = tbt & ~1
            hs = (hbt & 1) == 1
            ts = (tbt & 1) == 1
            u = []
            rh = []
            for j in range(GRPS):
                col = lane + j * LANES
                hx = plsc.load_gather(h_v, [bcast, hb + col])
                tx = plsc.load_gather(t_v, [bcast, tb + col])
                hbf = plsc.bitcast(hx, jnp.bfloat16)
                tbf = plsc.bitcast(tx, jnp.bfloat16)
                hlo, hhi = plsc.unpack(hbf, format=plsc.PackFormat.INTERLEAVED)
                tlo, thi = plsc.unpack(tbf, format=plsc.PackFormat.INTERLEAVED)
                h_j = jnp.where(hs, hhi, hlo)
                t_j = jnp.where(ts, thi, tlo)
                u.append(h_j - t_j)
                rh.append(rel_v[i, pl.ds(DIM + j * LANES, LANES)])
            acc = u[0] * rh[0]
            for j in range(1, GRPS):
                acc = acc + u[j] * rh[j]
            for p in perms:
                acc = acc + acc.at[p].get(mode="promise_in_bounds")
            for j in range(GRPS):
                sl = pl.ds(j * LANES, LANES)
                out_v[i, sl] = u[j] + rel_v[i, sl] - rh[j] * acc
            return 0

        lax.fori_loop(0, CHUNK, row, 0)
        pltpu.sync_copy(out_v, out_hbm.at[pl.ds(base, CHUNK)])


@jax.jit
def kernel(head, relation, tail, ent_emb, rel_emb, rel_hyper):
    ent2 = ent_emb.astype(jnp.bfloat16).reshape(500000, 2 * DIM)
    rel_cat = jnp.concatenate([rel_emb, rel_hyper], axis=1)
    mesh = plsc.VectorSubcoreMesh(core_axis_name="c", subcore_axis_name="s")
    run = functools.partial(
        pl.kernel,
        mesh=mesh,
        out_type=jax.ShapeDtypeStruct((BATCH, DIM), jnp.float32),
        scratch_types=[
            pltpu.VMEM((NK, KIDX), jnp.int32),
            pltpu.VMEM((NK, KIDX), jnp.int32),
            pltpu.VMEM((NK, KIDX), jnp.int32),
            pltpu.VMEM((CHUNK,), jnp.int32),
            pltpu.VMEM((CHUNK,), jnp.int32),
            pltpu.VMEM((CHUNK, 2 * DIM), jnp.float32),
            pltpu.VMEM((CHUNK, 2 * DIM), jnp.float32),
            pltpu.VMEM((CHUNK, 2 * DIM), jnp.float32),
            pltpu.VMEM((CHUNK, DIM), jnp.float32),
            pltpu.SemaphoreType.DMA,
        ],
        compiler_params=pltpu.CompilerParams(use_tc_tiling_on_sc=True,
                                             needs_layout_passes=False),
    )(_tec_body)
    return run(head.astype(jnp.int32), relation.astype(jnp.int32),
               tail.astype(jnp.int32), ent2, rel_cat)


# speedup vs baseline: 1.0218x; 1.0076x over previous
import functools

import jax
import jax.numpy as jnp
from jax import lax
from jax.experimental import pallas as pl
from jax.experimental.pallas import tpu as pltpu
from jax.experimental.pallas import tpu_sc as plsc

BATCH = 16384
DIM = 64
LANES = 16
GRPS = DIM // LANES

_info = plsc.get_sparse_core_info()
NC, NS = _info.num_cores, _info.num_subcores
NW = NC * NS
PER_W = BATCH // NW
CHUNK = 256
NCHUNK = PER_W // CHUNK
KIDX = 128
NK = CHUNK // KIDX


def _tec_body(head_hbm, rel_hbm, tail_hbm, ent_hbm, rel_emb_hbm,
              rel_hyper_hbm, out_hbm,
              hidx, tidx, ridx, h_v, t_v, r_v, rh_v, sem):
    wid = lax.axis_index("s") * NC + lax.axis_index("c")
    wbase = wid * PER_W
    lane = lax.iota(jnp.int32, LANES)
    perms = [jnp.bitwise_xor(lane, s) for s in (1, 2, 4, 8)]

    for c in range(NCHUNK):
        base = wbase + c * CHUNK
        for k in range(NK):
            off = base + k * KIDX
            pltpu.sync_copy(head_hbm.at[pl.ds(off, KIDX)], hidx.at[k])
            pltpu.sync_copy(tail_hbm.at[pl.ds(off, KIDX)], tidx.at[k])
            pltpu.sync_copy(rel_hbm.at[pl.ds(off, KIDX)], ridx.at[k])
        cps = []
        for k in range(NK):
            dst = pl.ds(k * KIDX, KIDX)
            cps.append(pltpu.async_copy(ent_hbm.at[hidx.at[k]],
                                        h_v.at[dst], sem))
            cps.append(pltpu.async_copy(ent_hbm.at[tidx.at[k]],
                                        t_v.at[dst], sem))
            cps.append(pltpu.async_copy(rel_emb_hbm.at[ridx.at[k]],
                                        r_v.at[dst], sem))
            cps.append(pltpu.async_copy(rel_hyper_hbm.at[ridx.at[k]],
                                        rh_v.at[dst], sem))
        for cp in cps:
            cp.wait()

        def row(i, _):
            u = []
            rh = []
            for j in range(GRPS):
                sl = pl.ds(j * LANES, LANES)
                u.append(h_v[i, sl] - t_v[i, sl])
                rh.append(rh_v[i, sl])
            acc = u[0] * rh[0]
            for j in range(1, GRPS):
                acc = acc + u[j] * rh[j]
            for p in perms:
                acc = acc + acc.at[p].get(mode="promise_in_bounds")
            for j in range(GRPS):
                sl = pl.ds(j * LANES, LANES)
                h_v[i, sl] = u[j] + r_v[i, sl] - rh[j] * acc
            return 0

        lax.fori_loop(0, CHUNK, row, 0)
        pltpu.sync_copy(h_v, out_hbm.at[pl.ds(base, CHUNK)])


@jax.jit
def kernel(head, relation, tail, ent_emb, rel_emb, rel_hyper):
    mesh = plsc.VectorSubcoreMesh(core_axis_name="c", subcore_axis_name="s")
    run = functools.partial(
        pl.kernel,
        mesh=mesh,
        out_type=jax.ShapeDtypeStruct((BATCH, DIM), jnp.float32),
        scratch_types=[
            pltpu.VMEM((NK, KIDX), jnp.int32),
            pltpu.VMEM((NK, KIDX), jnp.int32),
            pltpu.VMEM((NK, KIDX), jnp.int32),
            pltpu.VMEM((CHUNK, DIM), jnp.float32),
            pltpu.VMEM((CHUNK, DIM), jnp.float32),
            pltpu.VMEM((CHUNK, DIM), jnp.float32),
            pltpu.VMEM((CHUNK, DIM), jnp.float32),
            pltpu.SemaphoreType.DMA,
        ],
        compiler_params=pltpu.CompilerParams(use_tc_tiling_on_sc=False),
    )(_tec_body)
    return run(head.astype(jnp.int32), relation.astype(jnp.int32),
               tail.astype(jnp.int32), ent_emb, rel_emb, rel_hyper)
